# baseline (speedup 1.0000x reference)
import functools

import jax
import jax.numpy as jnp
from jax import lax
from jax.experimental import pallas as pl
from jax.experimental.pallas import tpu as pltpu, tpu_sc as plsc

ATOM_DIMS = (119, 5, 12, 12, 10, 6, 6, 2, 2)
NF = len(ATOM_DIMS)
D = 128
NC, NS, L = 2, 16, 16
NW = NC * NS
B = 64


def _sc_lookup_sum(table, idx_cols, n_pad):
    rows_per_w = n_pad // NW
    n_chunks = rows_per_w // B
    mesh = plsc.VectorSubcoreMesh(core_axis_name="c", subcore_axis_name="s")

    @functools.partial(
        pl.kernel,
        out_type=jax.ShapeDtypeStruct((n_pad, D), jnp.float32),
        mesh=mesh,
        scratch_types=[
            pltpu.VMEM((NF, B), jnp.int32),
            pltpu.VMEM((NF, B, D), jnp.float32),
            pltpu.VMEM((B, D), jnp.float32),
            pltpu.SemaphoreType.DMA,
            pltpu.SemaphoreType.DMA,
        ],
    )
    def k(table_hbm, idx_hbm, out_hbm, idx_v, rows_v, out_v, sem_i, sem_g):
        wid = lax.axis_index("s") * NC + lax.axis_index("c")
        base0 = wid * rows_per_w

        def chunk_body(c, carry):
            base = base0 + c * B
            icopies = [
                pltpu.async_copy(idx_hbm.at[i, pl.ds(base, B)], idx_v.at[i], sem_i)
                for i in range(NF)
            ]
            for cp in icopies:
                cp.wait()
            gcopies = [
                pltpu.async_copy(table_hbm.at[idx_v.at[i]], rows_v.at[i], sem_g)
                for i in range(NF)
            ]
            for cp in gcopies:
                cp.wait()

            def row_body(r, rcarry):
                for j in range(D // L):
                    acc = rows_v[0, r, pl.ds(j * L, L)]
                    for i in range(1, NF):
                        acc = acc + rows_v[i, r, pl.ds(j * L, L)]
                    out_v[r, pl.ds(j * L, L)] = acc
                return rcarry

            lax.fori_loop(0, B, row_body, 0)
            pltpu.sync_copy(out_v, out_hbm.at[pl.ds(base, B)])
            return carry

        lax.fori_loop(0, n_chunks, chunk_body, 0)

    return k(table, idx_cols)


def kernel(x, pestat, W0, W1, W2, W3, W4, W5, W6, W7, W8):
    del pestat
    n = x.shape[0]
    n_pad = ((n + NW * B - 1) // (NW * B)) * (NW * B)
    table = jnp.concatenate([W0, W1, W2, W3, W4, W5, W6, W7, W8], axis=0)
    offs = jnp.array(
        [sum(ATOM_DIMS[:i]) for i in range(NF)], dtype=jnp.int32
    )
    idx = x.astype(jnp.int32) + offs[None, :]
    idx = jnp.pad(idx, ((0, n_pad - n), (0, 0)))
    idx_cols = idx.T.copy()
    out = _sc_lookup_sum(table, idx_cols, n_pad)
    return out[:n]


# speedup vs baseline: 5.3105x; 5.3105x over previous
import functools

import jax
import jax.numpy as jnp
from jax import lax
from jax.experimental import pallas as pl
from jax.experimental.pallas import tpu as pltpu, tpu_sc as plsc

D = 128
NC, NS, L = 2, 16, 16
NW = NC * NS
B = 64
NG = 4
TROWS = 704


def _build_group_tables(W0, W1, W2, W3, W4, W5, W6, W7, W8):
    def body(w0, w1, w2, w3, w4, w5, w6, w7, w8, o078, o12, o34, o56):
        p78 = (w7[:][:, None, :] + w8[:][None, :, :]).reshape(4, D)
        o078[...] = (w0[:][:, None, :] + p78[None, :, :]).reshape(476, D)
        o12[...] = (w1[:][:, None, :] + w2[:][None, :, :]).reshape(60, D)
        o34[...] = (w3[:][:, None, :] + w4[:][None, :, :]).reshape(120, D)
        o56[...] = (w5[:][:, None, :] + w6[:][None, :, :]).reshape(36, D)

    return pl.pallas_call(
        body,
        out_shape=[
            jax.ShapeDtypeStruct((476, D), jnp.float32),
            jax.ShapeDtypeStruct((60, D), jnp.float32),
            jax.ShapeDtypeStruct((120, D), jnp.float32),
            jax.ShapeDtypeStruct((36, D), jnp.float32),
        ],
    )(W0, W1, W2, W3, W4, W5, W6, W7, W8)


def _sc_lookup_sum(table_flat, idx_flat, n_pad):
    rows_per_w = n_pad // NW
    n_outer = rows_per_w // (2 * B)
    ipw = NG * rows_per_w
    mesh = plsc.VectorSubcoreMesh(
        core_axis_name="c", subcore_axis_name="s", num_cores=NC, num_subcores=NS
    )

    @functools.partial(
        pl.kernel,
        out_type=jax.ShapeDtypeStruct((n_pad * D,), jnp.float32),
        mesh=mesh,
        compiler_params=pltpu.CompilerParams(needs_layout_passes=False),
        scratch_types=[
            pltpu.VMEM((TROWS * D,), jnp.float32),
            pltpu.VMEM((ipw,), jnp.int32),
            pltpu.VMEM((2, B * D), jnp.float32),
            pltpu.SemaphoreType.DMA,
            pltpu.SemaphoreType.DMA,
        ],
    )
    def k(table_hbm, idx_hbm, out_hbm, table_v, idx_v, out_v, sem0, sem1):
        wid = lax.axis_index("s") * NC + lax.axis_index("c")
        tcp = pltpu.async_copy(table_hbm, table_v, sem0)
        icp = pltpu.async_copy(idx_hbm.at[pl.ds(wid * ipw, ipw)], idx_v, sem1)
        tcp.wait()
        icp.wait()
        sems = (sem0, sem1)
        base_w = wid * rows_per_w

        def outer(t, carry):
            for p in range(2):
                local = (2 * t + p) * B

                def blk(b, rc):
                    ivs = [
                        idx_v[pl.ds(g * rows_per_w + local + b * L, L)] * D
                        for g in range(NG)
                    ]
                    for rr in range(L):
                        rows = [
                            jnp.full((L,), ivs[g][rr], jnp.int32) for g in range(NG)
                        ]
                        for j in range(D // L):
                            col = lax.iota(jnp.int32, L) + j * L
                            acc = plsc.load_gather(table_v, [rows[0] + col])
                            for g in range(1, NG):
                                acc = acc + plsc.load_gather(table_v, [rows[g] + col])
                            out_v[p, pl.ds((b * L + rr) * D + j * L, L)] = acc
                    return rc

                lax.fori_loop(0, B // L, blk, 0)
                pltpu.sync_copy(
                    out_v.at[p],
                    out_hbm.at[pl.ds((base_w + local) * D, B * D)],
                )
            return carry

        lax.fori_loop(0, n_outer, outer, 0)

    return k(table_flat, idx_flat)


def kernel(x, pestat, W0, W1, W2, W3, W4, W5, W6, W7, W8):
    del pestat
    n = x.shape[0]
    n_pad = ((n + NW * B - 1) // (NW * B)) * (NW * B)

    o078, o12, o34, o56 = _build_group_tables(W0, W1, W2, W3, W4, W5, W6, W7, W8)
    zpad = jnp.zeros((4, D), dtype=jnp.float32)
    table = jnp.concatenate([o078, zpad, o12, zpad, o34, o56, zpad], axis=0)

    xi = x.astype(jnp.int32)
    gi = jnp.stack(
        [
            xi[:, 0] * 4 + xi[:, 7] * 2 + xi[:, 8],
            480 + xi[:, 1] * 12 + xi[:, 2],
            544 + xi[:, 3] * 10 + xi[:, 4],
            664 + xi[:, 5] * 6 + xi[:, 6],
        ],
        axis=1,
    )
    gi = jnp.pad(gi, ((0, n_pad - n), (0, 0)))
    rows_per_w = n_pad // NW
    idx_flat = gi.reshape(NW, rows_per_w, NG).transpose(0, 2, 1).reshape(-1)

    out = _sc_lookup_sum(table.reshape(-1), idx_flat, n_pad)
    return out.reshape(n_pad, D)[:n]
